# baseline (speedup 1.0000x reference)
import functools

import jax
import jax.numpy as jnp
from jax import lax
from jax.experimental import pallas as pl
from jax.experimental.pallas import tpu as pltpu, tpu_sc as plsc

N = 10000
NP = 10112
E = 320000
D = 128
G = 64
NC = 2
NS = 16
NW = NC * NS
K = 125
KE = 125
NCH = E // NW // KE
GR = 8
RT = NP // NS
BW = 16
BC = N // BW // K

_MESH = plsc.VectorSubcoreMesh(core_axis_name="c", subcore_axis_name="s")


@functools.partial(
    pl.kernel,
    out_type=jax.ShapeDtypeStruct((NC, NP, D), jnp.float32),
    mesh=_MESH,
    scratch_types=[
        pltpu.VMEM((GR, KE), jnp.int32),
        pltpu.VMEM((KE, D), jnp.float32),
        pltpu.VMEM_SHARED((NP, D), jnp.float32),
    ],
)
def _sc_hist(dst3, degp_out, didx, ones_v, dacc):
    c = lax.axis_index("c")
    s = lax.axis_index("s")
    wid = c * NS + s

    zv = jnp.zeros((16,), jnp.float32)
    def zb(i, _):
        for cc in range(D // 16):
            ones_v[i, pl.ds(cc * 16, 16)] = zv
        return _
    lax.fori_loop(0, KE, zb, None)
    def zc(k, _):
        pltpu.sync_copy(ones_v, dacc.at[pl.ds(s * RT + k * KE, KE)])
        return _
    lax.fori_loop(0, RT // KE, zc, None)
    pltpu.sync_copy(ones_v.at[pl.ds(0, RT % KE)],
                    dacc.at[pl.ds(s * RT + (RT // KE) * KE, RT % KE)])
    e0 = jnp.where(lax.iota(jnp.int32, 16) == 0,
                   jnp.float32(1), jnp.float32(0))
    def ob(i, _):
        ones_v[i, pl.ds(0, 16)] = e0
        return _
    lax.fori_loop(0, KE, ob, None)
    plsc.subcore_barrier()

    def group(g, _):
        pltpu.sync_copy(dst3.at[wid, pl.ds(g * GR, GR)], didx)
        def chunk(j, _):
            pltpu.sync_copy(ones_v, dacc.at[didx.at[j]], add=True)
            return _
        lax.fori_loop(0, GR, chunk, None)
        return _
    lax.fori_loop(0, NCH // GR, group, None)

    plsc.subcore_barrier()
    pltpu.sync_copy(dacc.at[pl.ds(s * RT, RT)],
                    degp_out.at[c, pl.ds(s * RT, RT)])


@functools.partial(
    pl.kernel,
    out_type=jax.ShapeDtypeStruct((NC, NP, D), jnp.float32),
    mesh=_MESH,
    scratch_types=[
        pltpu.VMEM((GR, KE), jnp.int32),
        pltpu.VMEM((GR, KE), jnp.int32),
        pltpu.VMEM((KE, D), jnp.float32),
        pltpu.VMEM((KE, D), jnp.float32),
        pltpu.SemaphoreType.DMA,
        pltpu.SemaphoreType.DMA,
        pltpu.VMEM_SHARED((NP, D), jnp.float32),
    ],
)
def _sc_edge_scatter(hs, src3, dst3, s_out,
                     sidx, didx, buf0, buf1, sem0, sem1, acc):
    c = lax.axis_index("c")
    s = lax.axis_index("s")
    wid = c * NS + s

    zv = jnp.zeros((16,), jnp.float32)
    def zb(i, _):
        for cc in range(D // 16):
            buf0[i, pl.ds(cc * 16, 16)] = zv
        return _
    lax.fori_loop(0, KE, zb, None)
    def zc(k, _):
        pltpu.sync_copy(buf0, acc.at[pl.ds(s * RT + k * KE, KE)])
        return _
    lax.fori_loop(0, RT // KE, zc, None)
    pltpu.sync_copy(buf0.at[pl.ds(0, RT % KE)],
                    acc.at[pl.ds(s * RT + (RT // KE) * KE, RT % KE)])
    plsc.subcore_barrier()

    def group(g, _):
        pltpu.sync_copy(src3.at[wid, pl.ds(g * GR, GR)], sidx)
        pltpu.sync_copy(dst3.at[wid, pl.ds(g * GR, GR)], didx)
        def chunk(j, _):
            pltpu.async_copy(hs.at[sidx.at[j]], buf0, sem0).wait()
            pltpu.sync_copy(buf0, acc.at[didx.at[j]], add=True)
            return _
        lax.fori_loop(0, GR, chunk, None)
        return _
    lax.fori_loop(0, NCH // GR, group, None)

    plsc.subcore_barrier()
    pltpu.sync_copy(acc.at[pl.ds(s * RT, RT)],
                    s_out.at[c, pl.ds(s * RT, RT)])


@functools.partial(
    pl.kernel,
    out_type=jax.ShapeDtypeStruct((NC, G, D), jnp.float32),
    mesh=_MESH,
    scratch_types=[
        pltpu.VMEM((BC, K), jnp.int32),
        pltpu.VMEM((BC, K), jnp.int32),
        pltpu.VMEM((K, D), jnp.float32),
        pltpu.SemaphoreType.DMA,
        pltpu.VMEM_SHARED((G, D), jnp.float32),
    ],
)
def _sc_pool(h2, bidx, ridx, zg, p_out, bv, rv, rbuf, sem, pacc):
    c = lax.axis_index("c")
    s = lax.axis_index("s")
    pw = c * (BW // NC) + s

    @pl.when(s == 0)
    def _():
        pltpu.sync_copy(zg, pacc)
    plsc.subcore_barrier()

    @pl.when(s < BW // NC)
    def _():
        pltpu.sync_copy(bidx.at[pw], bv)
        pltpu.sync_copy(ridx.at[pw], rv)
        def body(j, _):
            pltpu.async_copy(h2.at[rv.at[j]], rbuf, sem).wait()
            pltpu.sync_copy(rbuf, pacc.at[bv.at[j]], add=True)
            return _
        lax.fori_loop(0, BC, body, None)

    plsc.subcore_barrier()
    @pl.when(s == 0)
    def _():
        pltpu.sync_copy(pacc, p_out.at[c])


_RB = 2000


def _dinv_of(d_ref):
    deg = d_ref[0, :, 0] + d_ref[1, :, 0] + 1.0
    return lax.rsqrt(deg)


def _mm_scale_body(x_ref, w_ref, d_ref, o_ref):
    dinv = _dinv_of(d_ref)
    h = jnp.dot(x_ref[...], w_ref[...], preferred_element_type=jnp.float32)
    o_ref[...] = h * dinv[:, None]


def _mm_scale(x, W, degp):
    return pl.pallas_call(
        _mm_scale_body,
        grid=(N // _RB,),
        in_specs=[pl.BlockSpec((_RB, D), lambda i: (i, 0)),
                  pl.BlockSpec((D, D), lambda i: (0, 0)),
                  pl.BlockSpec((NC, _RB, 16), lambda i: (0, i, 0))],
        out_specs=pl.BlockSpec((_RB, D), lambda i: (i, 0)),
        out_shape=jax.ShapeDtypeStruct((N, D), jnp.float32),
    )(x, W, degp)


def _fuse_body(s_ref, hs_ref, d_ref, b_ref, w_ref, o_ref):
    dinv = _dinv_of(d_ref)
    t = (s_ref[0] + s_ref[1] + hs_ref[...]) * dinv[:, None] + b_ref[...]
    h1 = jnp.maximum(t, 0.0)
    o_ref[...] = jnp.dot(h1, w_ref[...],
                         preferred_element_type=jnp.float32) * dinv[:, None]


def _fuse(S, hs, degp, b, W):
    return pl.pallas_call(
        _fuse_body,
        grid=(N // _RB,),
        in_specs=[pl.BlockSpec((NC, _RB, D), lambda i: (0, i, 0)),
                  pl.BlockSpec((_RB, D), lambda i: (i, 0)),
                  pl.BlockSpec((NC, _RB, 16), lambda i: (0, i, 0)),
                  pl.BlockSpec((1, D), lambda i: (0, 0)),
                  pl.BlockSpec((D, D), lambda i: (0, 0))],
        out_specs=pl.BlockSpec((_RB, D), lambda i: (i, 0)),
        out_shape=jax.ShapeDtypeStruct((N, D), jnp.float32),
    )(S, hs, degp, b, W)


def _epi_body(s_ref, hs_ref, d_ref, b_ref, o_ref):
    dinv = _dinv_of(d_ref)
    t = (s_ref[0] + s_ref[1] + hs_ref[...]) * dinv[:, None] + b_ref[...]
    o_ref[...] = jnp.maximum(t, 0.0)


def _epi(S, hs, degp, b):
    return pl.pallas_call(
        _epi_body,
        grid=(N // _RB,),
        in_specs=[pl.BlockSpec((NC, _RB, D), lambda i: (0, i, 0)),
                  pl.BlockSpec((_RB, D), lambda i: (i, 0)),
                  pl.BlockSpec((NC, _RB, 16), lambda i: (0, i, 0)),
                  pl.BlockSpec((1, D), lambda i: (0, 0))],
        out_specs=pl.BlockSpec((_RB, D), lambda i: (i, 0)),
        out_shape=jax.ShapeDtypeStruct((N, D), jnp.float32),
    )(S, hs, degp, b)


def _head_body(p_ref, b_ref2, w1_ref, b1_ref, w2_ref, b2_ref, w3_ref, b3_ref,
               emb_ref, out_ref):
    b = b_ref2[...]
    gids = lax.broadcasted_iota(jnp.int32, (G, 1, 1), 0)
    cnt = jnp.sum((b[None, :, :] == gids).astype(jnp.float32), axis=(1, 2))
    g = (p_ref[0] + p_ref[1]) / jnp.maximum(cnt, 1.0)[:, None]
    a = jnp.maximum(
        jnp.dot(g, w1_ref[...], preferred_element_type=jnp.float32)
        + b1_ref[...], 0.0)
    emb = jnp.maximum(
        jnp.dot(a, w2_ref[...], preferred_element_type=jnp.float32)
        + b2_ref[...], 0.0)
    emb_ref[...] = emb
    out_ref[...] = jnp.dot(emb, w3_ref[...],
                           preferred_element_type=jnp.float32) + b3_ref[...]


def _head(P, batch2d, fc1_W, fc1_b, fce_W, fce_b, fco_W, fco_b):
    return pl.pallas_call(
        _head_body,
        out_shape=[jax.ShapeDtypeStruct((G, D), jnp.float32),
                   jax.ShapeDtypeStruct((G, 1), jnp.float32)],
    )(P, batch2d, fc1_W, fc1_b, fce_W, fce_b, fco_W, fco_b)


def kernel(x, edge_index, batch, W1, b1, W2, b2,
           fc1_W, fc1_b, fce_W, fce_b, fco_W, fco_b):
    ei = edge_index.astype(jnp.int32)
    src3 = ei[0].reshape(NW, NCH, KE)
    dst3 = ei[1].reshape(NW, NCH, KE)
    bidx = batch.astype(jnp.int32).reshape(BW, BC, K)
    ridx = jnp.arange(N, dtype=jnp.int32).reshape(BW, BC, K)

    zg = jnp.zeros((G, D), jnp.float32)

    degp = _sc_hist(dst3)[:, :, :16]

    hs1 = _mm_scale(x, W1, degp)
    S1 = _sc_edge_scatter(hs1, src3, dst3)
    hs2 = _fuse(S1, hs1, degp, b1.reshape(1, D), W2)
    S2 = _sc_edge_scatter(hs2, src3, dst3)
    h2 = _epi(S2, hs2, degp, b2.reshape(1, D))

    P = _sc_pool(h2, bidx, ridx, zg)
    batch2d = batch.astype(jnp.int32).reshape(80, K)
    emb, out = _head(P, batch2d, fc1_W, fc1_b.reshape(1, -1),
                     fce_W, fce_b.reshape(1, -1),
                     fco_W, fco_b.reshape(1, 1))
    return emb, out


# speedup vs baseline: 1.2714x; 1.2714x over previous
import functools

import jax
import jax.numpy as jnp
from jax import lax
from jax.experimental import pallas as pl
from jax.experimental.pallas import tpu as pltpu, tpu_sc as plsc

N = 10000
NP = 10112
E = 320000
D = 128
G = 64
NC = 2
NS = 16
NW = NC * NS
K = 125
KE = 125
NCH = E // NW // KE
GR = 8
RT = NP // NS
BW = 16
BC = N // BW // K

_MESH = plsc.VectorSubcoreMesh(core_axis_name="c", subcore_axis_name="s")


@functools.partial(
    pl.kernel,
    out_type=jax.ShapeDtypeStruct((NC, NP, D), jnp.float32),
    mesh=_MESH,
    scratch_types=[
        pltpu.VMEM((GR, KE), jnp.int32),
        pltpu.VMEM((KE, D), jnp.float32),
        pltpu.VMEM_SHARED((NP, D), jnp.float32),
    ],
)
def _sc_hist(dst3, degp_out, didx, ones_v, dacc):
    c = lax.axis_index("c")
    s = lax.axis_index("s")
    wid = c * NS + s

    zv = jnp.zeros((16,), jnp.float32)
    def zb(i, _):
        for cc in range(D // 16):
            ones_v[i, pl.ds(cc * 16, 16)] = zv
        return _
    lax.fori_loop(0, KE, zb, None)
    def zc(k, _):
        pltpu.sync_copy(ones_v, dacc.at[pl.ds(s * RT + k * KE, KE)])
        return _
    lax.fori_loop(0, RT // KE, zc, None)
    pltpu.sync_copy(ones_v.at[pl.ds(0, RT % KE)],
                    dacc.at[pl.ds(s * RT + (RT // KE) * KE, RT % KE)])
    e0 = jnp.where(lax.iota(jnp.int32, 16) == 0,
                   jnp.float32(1), jnp.float32(0))
    def ob(i, _):
        ones_v[i, pl.ds(0, 16)] = e0
        return _
    lax.fori_loop(0, KE, ob, None)
    plsc.subcore_barrier()

    def group(g, _):
        pltpu.sync_copy(dst3.at[wid, pl.ds(g * GR, GR)], didx)
        def chunk(j, _):
            pltpu.sync_copy(ones_v, dacc.at[didx.at[j]], add=True)
            return _
        lax.fori_loop(0, GR, chunk, None)
        return _
    lax.fori_loop(0, NCH // GR, group, None)

    plsc.subcore_barrier()
    pltpu.sync_copy(dacc.at[pl.ds(s * RT, RT)],
                    degp_out.at[c, pl.ds(s * RT, RT)])


@functools.partial(
    pl.kernel,
    out_type=jax.ShapeDtypeStruct((NC, NP, D), jnp.float32),
    mesh=_MESH,
    scratch_types=[
        pltpu.VMEM((GR, KE), jnp.int32),
        pltpu.VMEM((GR, KE), jnp.int32),
        pltpu.VMEM((KE, D), jnp.float32),
        pltpu.VMEM((KE, D), jnp.float32),
        pltpu.SemaphoreType.DMA,
        pltpu.SemaphoreType.DMA,
        pltpu.VMEM_SHARED((NP, D), jnp.float32),
    ],
)
def _sc_edge_scatter(hs, src3, dst3, s_out,
                     sidx, didx, buf0, buf1, sem0, sem1, acc):
    c = lax.axis_index("c")
    s = lax.axis_index("s")
    wid = c * NS + s

    zv = jnp.zeros((16,), jnp.float32)
    def zb(i, _):
        for cc in range(D // 16):
            buf0[i, pl.ds(cc * 16, 16)] = zv
        return _
    lax.fori_loop(0, KE, zb, None)
    def zc(k, _):
        pltpu.sync_copy(buf0, acc.at[pl.ds(s * RT + k * KE, KE)])
        return _
    lax.fori_loop(0, RT // KE, zc, None)
    pltpu.sync_copy(buf0.at[pl.ds(0, RT % KE)],
                    acc.at[pl.ds(s * RT + (RT // KE) * KE, RT % KE)])
    plsc.subcore_barrier()

    def group(g, _):
        pltpu.sync_copy(src3.at[wid, pl.ds(g * GR, GR)], sidx)
        pltpu.sync_copy(dst3.at[wid, pl.ds(g * GR, GR)], didx)
        bufs = (buf0, buf1)
        sems = (sem0, sem1)
        cps = [pltpu.async_copy(hs.at[sidx.at[0]], buf0, sem0), None]
        for j in range(GR):
            if j + 1 < GR:
                cps[(j + 1) % 2] = pltpu.async_copy(
                    hs.at[sidx.at[j + 1]], bufs[(j + 1) % 2],
                    sems[(j + 1) % 2])
            cps[j % 2].wait()
            pltpu.sync_copy(bufs[j % 2], acc.at[didx.at[j]], add=True)
        return _
    lax.fori_loop(0, NCH // GR, group, None)

    plsc.subcore_barrier()
    pltpu.sync_copy(acc.at[pl.ds(s * RT, RT)],
                    s_out.at[c, pl.ds(s * RT, RT)])


@functools.partial(
    pl.kernel,
    out_type=jax.ShapeDtypeStruct((NC, G, D), jnp.float32),
    mesh=_MESH,
    scratch_types=[
        pltpu.VMEM((BC, K), jnp.int32),
        pltpu.VMEM((BC, K), jnp.int32),
        pltpu.VMEM((K, D), jnp.float32),
        pltpu.SemaphoreType.DMA,
        pltpu.VMEM_SHARED((G, D), jnp.float32),
    ],
)
def _sc_pool(h2, bidx, ridx, zg, p_out, bv, rv, rbuf, sem, pacc):
    c = lax.axis_index("c")
    s = lax.axis_index("s")
    pw = c * (BW // NC) + s

    @pl.when(s == 0)
    def _():
        pltpu.sync_copy(zg, pacc)
    plsc.subcore_barrier()

    @pl.when(s < BW // NC)
    def _():
        pltpu.sync_copy(bidx.at[pw], bv)
        pltpu.sync_copy(ridx.at[pw], rv)
        def body(j, _):
            pltpu.async_copy(h2.at[rv.at[j]], rbuf, sem).wait()
            pltpu.sync_copy(rbuf, pacc.at[bv.at[j]], add=True)
            return _
        lax.fori_loop(0, BC, body, None)

    plsc.subcore_barrier()
    @pl.when(s == 0)
    def _():
        pltpu.sync_copy(pacc, p_out.at[c])


_RB = 2000


def _dinv_of(d_ref):
    deg = d_ref[0, :, 0] + d_ref[1, :, 0] + 1.0
    return lax.rsqrt(deg)


def _mm_scale_body(x_ref, w_ref, d_ref, o_ref):
    dinv = _dinv_of(d_ref)
    h = jnp.dot(x_ref[...], w_ref[...], preferred_element_type=jnp.float32)
    o_ref[...] = h * dinv[:, None]


def _mm_scale(x, W, degp):
    return pl.pallas_call(
        _mm_scale_body,
        grid=(N // _RB,),
        in_specs=[pl.BlockSpec((_RB, D), lambda i: (i, 0)),
                  pl.BlockSpec((D, D), lambda i: (0, 0)),
                  pl.BlockSpec((NC, _RB, 16), lambda i: (0, i, 0))],
        out_specs=pl.BlockSpec((_RB, D), lambda i: (i, 0)),
        out_shape=jax.ShapeDtypeStruct((N, D), jnp.float32),
    )(x, W, degp)


def _fuse_body(s_ref, hs_ref, d_ref, b_ref, w_ref, o_ref):
    dinv = _dinv_of(d_ref)
    t = (s_ref[0] + s_ref[1] + hs_ref[...]) * dinv[:, None] + b_ref[...]
    h1 = jnp.maximum(t, 0.0)
    o_ref[...] = jnp.dot(h1, w_ref[...],
                         preferred_element_type=jnp.float32) * dinv[:, None]


def _fuse(S, hs, degp, b, W):
    return pl.pallas_call(
        _fuse_body,
        grid=(N // _RB,),
        in_specs=[pl.BlockSpec((NC, _RB, D), lambda i: (0, i, 0)),
                  pl.BlockSpec((_RB, D), lambda i: (i, 0)),
                  pl.BlockSpec((NC, _RB, 16), lambda i: (0, i, 0)),
                  pl.BlockSpec((1, D), lambda i: (0, 0)),
                  pl.BlockSpec((D, D), lambda i: (0, 0))],
        out_specs=pl.BlockSpec((_RB, D), lambda i: (i, 0)),
        out_shape=jax.ShapeDtypeStruct((N, D), jnp.float32),
    )(S, hs, degp, b, W)


def _epi_body(s_ref, hs_ref, d_ref, b_ref, o_ref):
    dinv = _dinv_of(d_ref)
    t = (s_ref[0] + s_ref[1] + hs_ref[...]) * dinv[:, None] + b_ref[...]
    o_ref[...] = jnp.maximum(t, 0.0)


def _epi(S, hs, degp, b):
    return pl.pallas_call(
        _epi_body,
        grid=(N // _RB,),
        in_specs=[pl.BlockSpec((NC, _RB, D), lambda i: (0, i, 0)),
                  pl.BlockSpec((_RB, D), lambda i: (i, 0)),
                  pl.BlockSpec((NC, _RB, 16), lambda i: (0, i, 0)),
                  pl.BlockSpec((1, D), lambda i: (0, 0))],
        out_specs=pl.BlockSpec((_RB, D), lambda i: (i, 0)),
        out_shape=jax.ShapeDtypeStruct((N, D), jnp.float32),
    )(S, hs, degp, b)


def _head_body(p_ref, b_ref2, w1_ref, b1_ref, w2_ref, b2_ref, w3_ref, b3_ref,
               emb_ref, out_ref):
    b = b_ref2[...]
    gids = lax.broadcasted_iota(jnp.int32, (G, 1, 1), 0)
    cnt = jnp.sum((b[None, :, :] == gids).astype(jnp.float32), axis=(1, 2))
    g = (p_ref[0] + p_ref[1]) / jnp.maximum(cnt, 1.0)[:, None]
    a = jnp.maximum(
        jnp.dot(g, w1_ref[...], preferred_element_type=jnp.float32)
        + b1_ref[...], 0.0)
    emb = jnp.maximum(
        jnp.dot(a, w2_ref[...], preferred_element_type=jnp.float32)
        + b2_ref[...], 0.0)
    emb_ref[...] = emb
    out_ref[...] = jnp.dot(emb, w3_ref[...],
                           preferred_element_type=jnp.float32) + b3_ref[...]


def _head(P, batch2d, fc1_W, fc1_b, fce_W, fce_b, fco_W, fco_b):
    return pl.pallas_call(
        _head_body,
        out_shape=[jax.ShapeDtypeStruct((G, D), jnp.float32),
                   jax.ShapeDtypeStruct((G, 1), jnp.float32)],
    )(P, batch2d, fc1_W, fc1_b, fce_W, fce_b, fco_W, fco_b)


def kernel(x, edge_index, batch, W1, b1, W2, b2,
           fc1_W, fc1_b, fce_W, fce_b, fco_W, fco_b):
    ei = edge_index.astype(jnp.int32)
    src3 = ei[0].reshape(NW, NCH, KE)
    dst3 = ei[1].reshape(NW, NCH, KE)
    bidx = batch.astype(jnp.int32).reshape(BW, BC, K)
    ridx = jnp.arange(N, dtype=jnp.int32).reshape(BW, BC, K)

    zg = jnp.zeros((G, D), jnp.float32)

    degp = _sc_hist(dst3)[:, :, :16]

    hs1 = _mm_scale(x, W1, degp)
    S1 = _sc_edge_scatter(hs1, src3, dst3)
    hs2 = _fuse(S1, hs1, degp, b1.reshape(1, D), W2)
    S2 = _sc_edge_scatter(hs2, src3, dst3)
    h2 = _epi(S2, hs2, degp, b2.reshape(1, D))

    P = _sc_pool(h2, bidx, ridx, zg)
    batch2d = batch.astype(jnp.int32).reshape(80, K)
    emb, out = _head(P, batch2d, fc1_W, fc1_b.reshape(1, -1),
                     fce_W, fce_b.reshape(1, -1),
                     fco_W, fco_b.reshape(1, 1))
    return emb, out


# speedup vs baseline: 1.2742x; 1.0022x over previous
import functools

import jax
import jax.numpy as jnp
from jax import lax
from jax.experimental import pallas as pl
from jax.experimental.pallas import tpu as pltpu, tpu_sc as plsc

N = 10000
NP = 10112
E = 320000
D = 128
G = 64
NC = 2
NS = 16
NW = NC * NS
K = 125
KE = 125
NCH = E // NW // KE
GR = 8
RT = NP // NS
BW = 16
BC = N // BW // K

_MESH = plsc.VectorSubcoreMesh(core_axis_name="c", subcore_axis_name="s")


@functools.partial(
    pl.kernel,
    out_type=jax.ShapeDtypeStruct((NC, NP, D), jnp.float32),
    mesh=_MESH,
    scratch_types=[
        pltpu.VMEM((GR, KE), jnp.int32),
        pltpu.VMEM((KE, D), jnp.float32),
        pltpu.SemaphoreType.DMA,
        pltpu.VMEM_SHARED((NP, D), jnp.float32),
    ],
)
def _sc_hist(dst3, degp_out, didx, ones_v, sem0, dacc):
    c = lax.axis_index("c")
    s = lax.axis_index("s")
    wid = c * NS + s

    zv = jnp.zeros((16,), jnp.float32)
    def zb(i, _):
        for cc in range(D // 16):
            ones_v[i, pl.ds(cc * 16, 16)] = zv
        return _
    lax.fori_loop(0, KE, zb, None)
    def zc(k, _):
        pltpu.sync_copy(ones_v, dacc.at[pl.ds(s * RT + k * KE, KE)])
        return _
    lax.fori_loop(0, RT // KE, zc, None)
    pltpu.sync_copy(ones_v.at[pl.ds(0, RT % KE)],
                    dacc.at[pl.ds(s * RT + (RT // KE) * KE, RT % KE)])
    e0 = jnp.where(lax.iota(jnp.int32, 16) == 0,
                   jnp.float32(1), jnp.float32(0))
    def ob(i, _):
        ones_v[i, pl.ds(0, 16)] = e0
        return _
    lax.fori_loop(0, KE, ob, None)
    plsc.subcore_barrier()

    def group(g, _):
        pltpu.sync_copy(dst3.at[wid, pl.ds(g * GR, GR)], didx)
        cps = [pltpu.async_copy(ones_v, dacc.at[didx.at[j]], sem0,
                                add=True)
               for j in range(GR)]
        for cp in cps:
            cp.wait()
        return _
    lax.fori_loop(0, NCH // GR, group, None)

    plsc.subcore_barrier()
    pltpu.sync_copy(dacc.at[pl.ds(s * RT, RT)],
                    degp_out.at[c, pl.ds(s * RT, RT)])


@functools.partial(
    pl.kernel,
    out_type=jax.ShapeDtypeStruct((NC, NP, D), jnp.float32),
    mesh=_MESH,
    scratch_types=[
        pltpu.VMEM((GR, KE), jnp.int32),
        pltpu.VMEM((GR, KE), jnp.int32),
        pltpu.VMEM((KE, D), jnp.float32),
        pltpu.VMEM((KE, D), jnp.float32),
        pltpu.SemaphoreType.DMA,
        pltpu.SemaphoreType.DMA,
        pltpu.VMEM_SHARED((NP, D), jnp.float32),
    ],
)
def _sc_edge_scatter(hs, src3, dst3, s_out,
                     sidx, didx, buf0, buf1, sem0, sem1, acc):
    c = lax.axis_index("c")
    s = lax.axis_index("s")
    wid = c * NS + s

    zv = jnp.zeros((16,), jnp.float32)
    def zb(i, _):
        for cc in range(D // 16):
            buf0[i, pl.ds(cc * 16, 16)] = zv
        return _
    lax.fori_loop(0, KE, zb, None)
    def zc(k, _):
        pltpu.sync_copy(buf0, acc.at[pl.ds(s * RT + k * KE, KE)])
        return _
    lax.fori_loop(0, RT // KE, zc, None)
    pltpu.sync_copy(buf0.at[pl.ds(0, RT % KE)],
                    acc.at[pl.ds(s * RT + (RT // KE) * KE, RT % KE)])
    plsc.subcore_barrier()

    def group(g, _):
        pltpu.sync_copy(src3.at[wid, pl.ds(g * GR, GR)], sidx)
        pltpu.sync_copy(dst3.at[wid, pl.ds(g * GR, GR)], didx)
        bufs = (buf0, buf1)
        sems = (sem0, sem1)
        cps = [pltpu.async_copy(hs.at[sidx.at[0]], buf0, sem0), None]
        for j in range(GR):
            if j + 1 < GR:
                cps[(j + 1) % 2] = pltpu.async_copy(
                    hs.at[sidx.at[j + 1]], bufs[(j + 1) % 2],
                    sems[(j + 1) % 2])
            cps[j % 2].wait()
            pltpu.sync_copy(bufs[j % 2], acc.at[didx.at[j]], add=True)
        return _
    lax.fori_loop(0, NCH // GR, group, None)

    plsc.subcore_barrier()
    pltpu.sync_copy(acc.at[pl.ds(s * RT, RT)],
                    s_out.at[c, pl.ds(s * RT, RT)])


@functools.partial(
    pl.kernel,
    out_type=jax.ShapeDtypeStruct((NC, G, D), jnp.float32),
    mesh=_MESH,
    scratch_types=[
        pltpu.VMEM((BC, K), jnp.int32),
        pltpu.VMEM((BC, K), jnp.int32),
        pltpu.VMEM((K, D), jnp.float32),
        pltpu.SemaphoreType.DMA,
        pltpu.VMEM_SHARED((G, D), jnp.float32),
    ],
)
def _sc_pool(h2, bidx, ridx, zg, p_out, bv, rv, rbuf, sem, pacc):
    c = lax.axis_index("c")
    s = lax.axis_index("s")
    pw = c * (BW // NC) + s

    @pl.when(s == 0)
    def _():
        pltpu.sync_copy(zg, pacc)
    plsc.subcore_barrier()

    @pl.when(s < BW // NC)
    def _():
        pltpu.sync_copy(bidx.at[pw], bv)
        pltpu.sync_copy(ridx.at[pw], rv)
        def body(j, _):
            pltpu.async_copy(h2.at[rv.at[j]], rbuf, sem).wait()
            pltpu.sync_copy(rbuf, pacc.at[bv.at[j]], add=True)
            return _
        lax.fori_loop(0, BC, body, None)

    plsc.subcore_barrier()
    @pl.when(s == 0)
    def _():
        pltpu.sync_copy(pacc, p_out.at[c])


_RB = 2000


def _dinv_of(d_ref):
    deg = d_ref[0, :, 0] + d_ref[1, :, 0] + 1.0
    return lax.rsqrt(deg)


def _mm_scale_body(x_ref, w_ref, d_ref, o_ref):
    dinv = _dinv_of(d_ref)
    h = jnp.dot(x_ref[...], w_ref[...], preferred_element_type=jnp.float32)
    o_ref[...] = h * dinv[:, None]


def _mm_scale(x, W, degp):
    return pl.pallas_call(
        _mm_scale_body,
        grid=(N // _RB,),
        in_specs=[pl.BlockSpec((_RB, D), lambda i: (i, 0)),
                  pl.BlockSpec((D, D), lambda i: (0, 0)),
                  pl.BlockSpec((NC, _RB, 16), lambda i: (0, i, 0))],
        out_specs=pl.BlockSpec((_RB, D), lambda i: (i, 0)),
        out_shape=jax.ShapeDtypeStruct((N, D), jnp.float32),
    )(x, W, degp)


def _fuse_body(s_ref, hs_ref, d_ref, b_ref, w_ref, o_ref):
    dinv = _dinv_of(d_ref)
    t = (s_ref[0] + s_ref[1] + hs_ref[...]) * dinv[:, None] + b_ref[...]
    h1 = jnp.maximum(t, 0.0)
    o_ref[...] = jnp.dot(h1, w_ref[...],
                         preferred_element_type=jnp.float32) * dinv[:, None]


def _fuse(S, hs, degp, b, W):
    return pl.pallas_call(
        _fuse_body,
        grid=(N // _RB,),
        in_specs=[pl.BlockSpec((NC, _RB, D), lambda i: (0, i, 0)),
                  pl.BlockSpec((_RB, D), lambda i: (i, 0)),
                  pl.BlockSpec((NC, _RB, 16), lambda i: (0, i, 0)),
                  pl.BlockSpec((1, D), lambda i: (0, 0)),
                  pl.BlockSpec((D, D), lambda i: (0, 0))],
        out_specs=pl.BlockSpec((_RB, D), lambda i: (i, 0)),
        out_shape=jax.ShapeDtypeStruct((N, D), jnp.float32),
    )(S, hs, degp, b, W)


def _epi_body(s_ref, hs_ref, d_ref, b_ref, o_ref):
    dinv = _dinv_of(d_ref)
    t = (s_ref[0] + s_ref[1] + hs_ref[...]) * dinv[:, None] + b_ref[...]
    o_ref[...] = jnp.maximum(t, 0.0)


def _epi(S, hs, degp, b):
    return pl.pallas_call(
        _epi_body,
        grid=(N // _RB,),
        in_specs=[pl.BlockSpec((NC, _RB, D), lambda i: (0, i, 0)),
                  pl.BlockSpec((_RB, D), lambda i: (i, 0)),
                  pl.BlockSpec((NC, _RB, 16), lambda i: (0, i, 0)),
                  pl.BlockSpec((1, D), lambda i: (0, 0))],
        out_specs=pl.BlockSpec((_RB, D), lambda i: (i, 0)),
        out_shape=jax.ShapeDtypeStruct((N, D), jnp.float32),
    )(S, hs, degp, b)


def _head_body(p_ref, b_ref2, w1_ref, b1_ref, w2_ref, b2_ref, w3_ref, b3_ref,
               emb_ref, out_ref):
    b = b_ref2[...]
    gids = lax.broadcasted_iota(jnp.int32, (G, 1, 1), 0)
    cnt = jnp.sum((b[None, :, :] == gids).astype(jnp.float32), axis=(1, 2))
    g = (p_ref[0] + p_ref[1]) / jnp.maximum(cnt, 1.0)[:, None]
    a = jnp.maximum(
        jnp.dot(g, w1_ref[...], preferred_element_type=jnp.float32)
        + b1_ref[...], 0.0)
    emb = jnp.maximum(
        jnp.dot(a, w2_ref[...], preferred_element_type=jnp.float32)
        + b2_ref[...], 0.0)
    emb_ref[...] = emb
    out_ref[...] = jnp.dot(emb, w3_ref[...],
                           preferred_element_type=jnp.float32) + b3_ref[...]


def _head(P, batch2d, fc1_W, fc1_b, fce_W, fce_b, fco_W, fco_b):
    return pl.pallas_call(
        _head_body,
        out_shape=[jax.ShapeDtypeStruct((G, D), jnp.float32),
                   jax.ShapeDtypeStruct((G, 1), jnp.float32)],
    )(P, batch2d, fc1_W, fc1_b, fce_W, fce_b, fco_W, fco_b)


def kernel(x, edge_index, batch, W1, b1, W2, b2,
           fc1_W, fc1_b, fce_W, fce_b, fco_W, fco_b):
    ei = edge_index.astype(jnp.int32)
    src3 = ei[0].reshape(NW, NCH, KE)
    dst3 = ei[1].reshape(NW, NCH, KE)
    bidx = batch.astype(jnp.int32).reshape(BW, BC, K)
    ridx = jnp.arange(N, dtype=jnp.int32).reshape(BW, BC, K)

    zg = jnp.zeros((G, D), jnp.float32)

    degp = _sc_hist(dst3)[:, :, :16]

    hs1 = _mm_scale(x, W1, degp)
    S1 = _sc_edge_scatter(hs1, src3, dst3)
    hs2 = _fuse(S1, hs1, degp, b1.reshape(1, D), W2)
    S2 = _sc_edge_scatter(hs2, src3, dst3)
    h2 = _epi(S2, hs2, degp, b2.reshape(1, D))

    P = _sc_pool(h2, bidx, ridx, zg)
    batch2d = batch.astype(jnp.int32).reshape(80, K)
    emb, out = _head(P, batch2d, fc1_W, fc1_b.reshape(1, -1),
                     fce_W, fce_b.reshape(1, -1),
                     fco_W, fco_b.reshape(1, 1))
    return emb, out


# speedup vs baseline: 1.2755x; 1.0010x over previous
import functools

import jax
import jax.numpy as jnp
from jax import lax
from jax.experimental import pallas as pl
from jax.experimental.pallas import tpu as pltpu, tpu_sc as plsc

N = 10000
NP = 10112
E = 320000
D = 128
G = 64
NC = 2
NS = 16
NW = NC * NS
K = 125
KE = 125
NCH = E // NW // KE
GR = 8
RT = NP // NS
BW = 16
BC = N // BW // K
HW = 128

_MESH = plsc.VectorSubcoreMesh(core_axis_name="c", subcore_axis_name="s")


@functools.partial(
    pl.kernel,
    out_type=jax.ShapeDtypeStruct((NC, NP, HW), jnp.float32),
    mesh=_MESH,
    scratch_types=[
        pltpu.VMEM((GR, KE), jnp.int32),
        pltpu.VMEM((KE, HW), jnp.float32),
        pltpu.SemaphoreType.DMA,
        pltpu.VMEM_SHARED((NP, HW), jnp.float32),
    ],
)
def _sc_hist(dst3, degp_out, didx, ones_v, sem0, dacc):
    c = lax.axis_index("c")
    s = lax.axis_index("s")
    wid = c * NS + s

    zv = jnp.zeros((16,), jnp.float32)
    def zb(i, _):
        for cc in range(HW // 16):
            ones_v[i, pl.ds(cc * 16, 16)] = zv
        return _
    lax.fori_loop(0, KE, zb, None)
    def zc(k, _):
        pltpu.sync_copy(ones_v, dacc.at[pl.ds(s * RT + k * KE, KE)])
        return _
    lax.fori_loop(0, RT // KE, zc, None)
    pltpu.sync_copy(ones_v.at[pl.ds(0, RT % KE)],
                    dacc.at[pl.ds(s * RT + (RT // KE) * KE, RT % KE)])
    e0 = jnp.where(lax.iota(jnp.int32, 16) == 0,
                   jnp.float32(1), jnp.float32(0))
    def ob(i, _):
        ones_v[i, pl.ds(0, 16)] = e0
        return _
    lax.fori_loop(0, KE, ob, None)
    plsc.subcore_barrier()

    def group(g, _):
        pltpu.sync_copy(dst3.at[wid, pl.ds(g * GR, GR)], didx)
        cps = [pltpu.async_copy(ones_v, dacc.at[didx.at[j]], sem0,
                                add=True)
               for j in range(GR)]
        for cp in cps:
            cp.wait()
        return _
    lax.fori_loop(0, NCH // GR, group, None)

    plsc.subcore_barrier()
    pltpu.sync_copy(dacc.at[pl.ds(s * RT, RT)],
                    degp_out.at[c, pl.ds(s * RT, RT)])


@functools.partial(
    pl.kernel,
    out_type=jax.ShapeDtypeStruct((NC, NP, D), jnp.float32),
    mesh=_MESH,
    scratch_types=[
        pltpu.VMEM((GR, KE), jnp.int32),
        pltpu.VMEM((GR, KE), jnp.int32),
        pltpu.VMEM((KE, D), jnp.float32),
        pltpu.VMEM((KE, D), jnp.float32),
        pltpu.SemaphoreType.DMA,
        pltpu.SemaphoreType.DMA,
        pltpu.VMEM_SHARED((NP, D), jnp.float32),
    ],
)
def _sc_edge_scatter(hs, src3, dst3, s_out,
                     sidx, didx, buf0, buf1, sem0, sem1, acc):
    c = lax.axis_index("c")
    s = lax.axis_index("s")
    wid = c * NS + s

    zv = jnp.zeros((16,), jnp.float32)
    def zb(i, _):
        for cc in range(D // 16):
            buf0[i, pl.ds(cc * 16, 16)] = zv
        return _
    lax.fori_loop(0, KE, zb, None)
    def zc(k, _):
        pltpu.sync_copy(buf0, acc.at[pl.ds(s * RT + k * KE, KE)])
        return _
    lax.fori_loop(0, RT // KE, zc, None)
    pltpu.sync_copy(buf0.at[pl.ds(0, RT % KE)],
                    acc.at[pl.ds(s * RT + (RT // KE) * KE, RT % KE)])
    plsc.subcore_barrier()

    def group(g, _):
        pltpu.sync_copy(src3.at[wid, pl.ds(g * GR, GR)], sidx)
        pltpu.sync_copy(dst3.at[wid, pl.ds(g * GR, GR)], didx)
        bufs = (buf0, buf1)
        sems = (sem0, sem1)
        cps = [pltpu.async_copy(hs.at[sidx.at[0]], buf0, sem0), None]
        for j in range(GR):
            if j + 1 < GR:
                cps[(j + 1) % 2] = pltpu.async_copy(
                    hs.at[sidx.at[j + 1]], bufs[(j + 1) % 2],
                    sems[(j + 1) % 2])
            cps[j % 2].wait()
            pltpu.sync_copy(bufs[j % 2], acc.at[didx.at[j]], add=True)
        return _
    lax.fori_loop(0, NCH // GR, group, None)

    plsc.subcore_barrier()
    pltpu.sync_copy(acc.at[pl.ds(s * RT, RT)],
                    s_out.at[c, pl.ds(s * RT, RT)])


@functools.partial(
    pl.kernel,
    out_type=jax.ShapeDtypeStruct((NC, G, D), jnp.float32),
    mesh=_MESH,
    scratch_types=[
        pltpu.VMEM((BC, K), jnp.int32),
        pltpu.VMEM((BC, K), jnp.int32),
        pltpu.VMEM((K, D), jnp.float32),
        pltpu.SemaphoreType.DMA,
        pltpu.VMEM_SHARED((G, D), jnp.float32),
    ],
)
def _sc_pool(h2, bidx, ridx, zg, p_out, bv, rv, rbuf, sem, pacc):
    c = lax.axis_index("c")
    s = lax.axis_index("s")
    pw = c * (BW // NC) + s

    @pl.when(s == 0)
    def _():
        pltpu.sync_copy(zg, pacc)
    plsc.subcore_barrier()

    @pl.when(s < BW // NC)
    def _():
        pltpu.sync_copy(bidx.at[pw], bv)
        pltpu.sync_copy(ridx.at[pw], rv)
        def body(j, _):
            pltpu.async_copy(h2.at[rv.at[j]], rbuf, sem).wait()
            pltpu.sync_copy(rbuf, pacc.at[bv.at[j]], add=True)
            return _
        lax.fori_loop(0, BC, body, None)

    plsc.subcore_barrier()
    @pl.when(s == 0)
    def _():
        pltpu.sync_copy(pacc, p_out.at[c])


_RB = 2000


def _dinv_of(d_ref):
    deg = d_ref[0, :, 0] + d_ref[1, :, 0] + 1.0
    return lax.rsqrt(deg)


def _mm_scale_body(x_ref, w_ref, d_ref, o_ref):
    dinv = _dinv_of(d_ref)
    h = jnp.dot(x_ref[...], w_ref[...], preferred_element_type=jnp.float32)
    o_ref[...] = h * dinv[:, None]


def _mm_scale(x, W, degp):
    return pl.pallas_call(
        _mm_scale_body,
        grid=(N // _RB,),
        in_specs=[pl.BlockSpec((_RB, D), lambda i: (i, 0)),
                  pl.BlockSpec((D, D), lambda i: (0, 0)),
                  pl.BlockSpec((NC, _RB, 16), lambda i: (0, i, 0))],
        out_specs=pl.BlockSpec((_RB, D), lambda i: (i, 0)),
        out_shape=jax.ShapeDtypeStruct((N, D), jnp.float32),
    )(x, W, degp)


def _fuse_body(s_ref, hs_ref, d_ref, b_ref, w_ref, o_ref):
    dinv = _dinv_of(d_ref)
    t = (s_ref[0] + s_ref[1] + hs_ref[...]) * dinv[:, None] + b_ref[...]
    h1 = jnp.maximum(t, 0.0)
    o_ref[...] = jnp.dot(h1, w_ref[...],
                         preferred_element_type=jnp.float32) * dinv[:, None]


def _fuse(S, hs, degp, b, W):
    return pl.pallas_call(
        _fuse_body,
        grid=(N // _RB,),
        in_specs=[pl.BlockSpec((NC, _RB, D), lambda i: (0, i, 0)),
                  pl.BlockSpec((_RB, D), lambda i: (i, 0)),
                  pl.BlockSpec((NC, _RB, 16), lambda i: (0, i, 0)),
                  pl.BlockSpec((1, D), lambda i: (0, 0)),
                  pl.BlockSpec((D, D), lambda i: (0, 0))],
        out_specs=pl.BlockSpec((_RB, D), lambda i: (i, 0)),
        out_shape=jax.ShapeDtypeStruct((N, D), jnp.float32),
    )(S, hs, degp, b, W)


def _epi_body(s_ref, hs_ref, d_ref, b_ref, o_ref):
    dinv = _dinv_of(d_ref)
    t = (s_ref[0] + s_ref[1] + hs_ref[...]) * dinv[:, None] + b_ref[...]
    o_ref[...] = jnp.maximum(t, 0.0)


def _epi(S, hs, degp, b):
    return pl.pallas_call(
        _epi_body,
        grid=(N // _RB,),
        in_specs=[pl.BlockSpec((NC, _RB, D), lambda i: (0, i, 0)),
                  pl.BlockSpec((_RB, D), lambda i: (i, 0)),
                  pl.BlockSpec((NC, _RB, 16), lambda i: (0, i, 0)),
                  pl.BlockSpec((1, D), lambda i: (0, 0))],
        out_specs=pl.BlockSpec((_RB, D), lambda i: (i, 0)),
        out_shape=jax.ShapeDtypeStruct((N, D), jnp.float32),
    )(S, hs, degp, b)


def _head_body(p_ref, b_ref2, w1_ref, b1_ref, w2_ref, b2_ref, w3_ref, b3_ref,
               emb_ref, out_ref):
    b = b_ref2[...]
    gids = lax.broadcasted_iota(jnp.int32, (G, 1, 1), 0)
    cnt = jnp.sum((b[None, :, :] == gids).astype(jnp.float32), axis=(1, 2))
    g = (p_ref[0] + p_ref[1]) / jnp.maximum(cnt, 1.0)[:, None]
    a = jnp.maximum(
        jnp.dot(g, w1_ref[...], preferred_element_type=jnp.float32)
        + b1_ref[...], 0.0)
    emb = jnp.maximum(
        jnp.dot(a, w2_ref[...], preferred_element_type=jnp.float32)
        + b2_ref[...], 0.0)
    emb_ref[...] = emb
    out_ref[...] = jnp.dot(emb, w3_ref[...],
                           preferred_element_type=jnp.float32) + b3_ref[...]


def _head(P, batch2d, fc1_W, fc1_b, fce_W, fce_b, fco_W, fco_b):
    return pl.pallas_call(
        _head_body,
        out_shape=[jax.ShapeDtypeStruct((G, D), jnp.float32),
                   jax.ShapeDtypeStruct((G, 1), jnp.float32)],
    )(P, batch2d, fc1_W, fc1_b, fce_W, fce_b, fco_W, fco_b)


def kernel(x, edge_index, batch, W1, b1, W2, b2,
           fc1_W, fc1_b, fce_W, fce_b, fco_W, fco_b):
    ei = edge_index.astype(jnp.int32)
    src3 = ei[0].reshape(NW, NCH, KE)
    dst3 = ei[1].reshape(NW, NCH, KE)
    bidx = batch.astype(jnp.int32).reshape(BW, BC, K)
    ridx = jnp.arange(N, dtype=jnp.int32).reshape(BW, BC, K)

    zg = jnp.zeros((G, D), jnp.float32)

    degp = _sc_hist(dst3)[:, :, :16]

    hs1 = _mm_scale(x, W1, degp)
    S1 = _sc_edge_scatter(hs1, src3, dst3)
    hs2 = _fuse(S1, hs1, degp, b1.reshape(1, D), W2)
    S2 = _sc_edge_scatter(hs2, src3, dst3)
    h2 = _epi(S2, hs2, degp, b2.reshape(1, D))

    P = _sc_pool(h2, bidx, ridx, zg)
    batch2d = batch.astype(jnp.int32).reshape(80, K)
    emb, out = _head(P, batch2d, fc1_W, fc1_b.reshape(1, -1),
                     fce_W, fce_b.reshape(1, -1),
                     fco_W, fco_b.reshape(1, 1))
    return emb, out
